# Initial kernel scaffold; baseline (speedup 1.0000x reference)
import jax
import jax.numpy as jnp
from jax.experimental import pallas as pl


def kernel(inp):
    raise NotImplementedError("write your pallas kernel here")


# speedup vs baseline: 1.7625x; 1.7625x over previous
import functools

import jax
import jax.numpy as jnp
from jax import lax
from jax.experimental import pallas as pl
from jax.experimental.pallas import tpu as pltpu
from jax.experimental.pallas import tpu_sc as plsc

NUM_SEGMENTS = 16
ROWS_PER_SEG = 2048
COLS = 1024

NC = 2
NS = 16
NW = NC * NS
CW = COLS // (NW // NUM_SEGMENTS)
NV = CW // 16
RC = 64
NCH = ROWS_PER_SEG // RC
UNROLL = 4


@functools.partial(
    pl.kernel,
    out_type=jax.ShapeDtypeStruct((NUM_SEGMENTS, COLS), jnp.float32),
    mesh=plsc.VectorSubcoreMesh(core_axis_name="c", subcore_axis_name="s"),
    scratch_types=[
        pltpu.VMEM((RC, CW), jnp.float32),
        pltpu.VMEM((RC, CW), jnp.float32),
        pltpu.VMEM((CW,), jnp.float32),
        pltpu.SemaphoreType.DMA,
        pltpu.SemaphoreType.DMA,
    ],
)
def _sc_segmean(inp_hbm, out_hbm, buf0, buf1, acc, sem0, sem1):
    wid = lax.axis_index("s") * NC + lax.axis_index("c")
    seg = wid // 2
    half = wid % 2
    row0 = seg * ROWS_PER_SEG
    col0 = half * CW

    bufs = (buf0, buf1)
    sems = (sem0, sem1)

    def start(k, b):
        pltpu.make_async_copy(
            inp_hbm.at[pl.ds(row0 + k * RC, RC), pl.ds(col0, CW)],
            bufs[b],
            sems[b],
        ).start()

    def wait(b):
        pltpu.make_async_copy(
            inp_hbm.at[pl.ds(row0, RC), pl.ds(col0, CW)],
            bufs[b],
            sems[b],
        ).wait()

    def accum(buf):
        def body(i, carry):
            r = i * UNROLL
            for j in range(NV):
                c = j * 16
                v = acc[pl.ds(c, 16)]
                for u in range(UNROLL):
                    v = v + buf[r + u, pl.ds(c, 16)]
                acc[pl.ds(c, 16)] = v
            return carry
        lax.fori_loop(0, RC // UNROLL, body, 0)

    start(0, 0)
    start(1, 1)
    zero = jnp.zeros((16,), jnp.float32)
    for j in range(NV):
        acc[pl.ds(j * 16, 16)] = zero

    def ring(i, carry):
        for b in range(2):
            k = i * 2 + b
            wait(b)
            accum(bufs[b])
            start(k + 2, b)
        return carry
    lax.fori_loop(0, (NCH - 2) // 2, ring, 0)

    wait(0)
    accum(buf0)
    wait(1)
    accum(buf1)

    scale = jnp.float32(1.0 / ROWS_PER_SEG)
    for j in range(NV):
        acc[pl.ds(j * 16, 16)] = acc[pl.ds(j * 16, 16)] * scale

    pltpu.sync_copy(acc, out_hbm.at[seg, pl.ds(col0, CW)])


def kernel(inp):
    return _sc_segmean(inp)
